# baseline (speedup 1.0000x reference)
import jax
import jax.numpy as jnp
from jax.experimental import pallas as pl

B = 32
D = 784
E = 8
H1 = 256
H2 = 128
O = 10


def _moe_body(x_ref, gate_W_ref, gate_b_ref, g_ref,
              W1_ref, b1_ref, W2_ref, b2_ref, W3_ref, b3_ref,
              final_ref, eout_ref, gate_ref, idx_ref):
    x = x_ref[...]
    logits = (jnp.dot(x, gate_W_ref[...], preferred_element_type=jnp.float32)
              + gate_b_ref[...])
    m = jnp.max(logits, axis=1, keepdims=True)
    ex = jnp.exp(logits - m)
    gate = ex / jnp.sum(ex, axis=1, keepdims=True)
    gate_ref[...] = gate
    z = jnp.log(gate + 1e-20) + g_ref[...]
    zm = jnp.max(z, axis=1, keepdims=True)
    cols = jax.lax.broadcasted_iota(jnp.int32, (B, E), 1)
    idx = jnp.min(jnp.where(z == zm, cols, E), axis=1, keepdims=True)
    idx_ref[...] = idx
    final = jnp.zeros((B, O), jnp.float32)
    for e in range(E):
        h1 = jnp.maximum(
            jnp.dot(x, W1_ref[e], preferred_element_type=jnp.float32)
            + b1_ref[e:e + 1, :], 0.0)
        h2 = jnp.maximum(
            jnp.dot(h1, W2_ref[e], preferred_element_type=jnp.float32)
            + b2_ref[e:e + 1, :], 0.0)
        oe = (jnp.dot(h2, W3_ref[e], preferred_element_type=jnp.float32)
              + b3_ref[e:e + 1, :])
        eout_ref[e] = oe
        final = final + jnp.where(idx == e, oe, 0.0)
    final_ref[...] = final


def kernel(x, gate_W, gate_b, W1, b1, W2, b2, W3, b3):
    x_flat = x.reshape(B, D)
    g = jax.random.gumbel(jax.random.key(42), (B, E), jnp.float32)
    final, eout, gate, idx = pl.pallas_call(
        _moe_body,
        out_shape=(
            jax.ShapeDtypeStruct((B, O), jnp.float32),
            jax.ShapeDtypeStruct((E, B, O), jnp.float32),
            jax.ShapeDtypeStruct((B, E), jnp.float32),
            jax.ShapeDtypeStruct((B, 1), jnp.int32),
        ),
    )(x_flat, gate_W, gate_b.reshape(1, E), g,
      W1, b1, W2, b2, W3, b3)
    return (final, eout.transpose(1, 0, 2), gate, idx.reshape(B))
